# baseline (speedup 1.0000x reference)
import functools

import jax
import jax.numpy as jnp
from jax import lax
from jax.experimental import pallas as pl
from jax.experimental.pallas import tpu as pltpu
from jax.experimental.pallas import tpu_sc as plsc

_NC = 2
_NS = 16
_NW = _NC * _NS
_CHUNK = 80
_SUP = 16


@functools.lru_cache(maxsize=None)
def _make_scatter(n_pad, n_feat, n_edges):
    edges_per_tile = n_edges // _NW
    n_chunks = edges_per_tile // _CHUNK
    n_sup = n_chunks // _SUP
    rows_per_tile = n_pad // _NS
    assert edges_per_tile * _NW == n_edges
    assert n_chunks * _CHUNK == edges_per_tile
    assert n_sup * _SUP == n_chunks
    assert rows_per_tile * _NS == n_pad and rows_per_tile % _CHUNK == 0

    mesh = plsc.VectorSubcoreMesh(core_axis_name="c", subcore_axis_name="s")

    @functools.partial(
        pl.kernel,
        mesh=mesh,
        out_type=jax.ShapeDtypeStruct((_NC, n_pad, n_feat), jnp.float32),
        scratch_types=[
            pltpu.VMEM_SHARED((n_pad, n_feat), jnp.float32),
            pltpu.VMEM((2, _SUP, _CHUNK), jnp.int32),
            pltpu.VMEM((2, _SUP, _CHUNK), jnp.int32),
            pltpu.VMEM((_CHUNK, n_feat), jnp.float32),
            pltpu.VMEM((_CHUNK, n_feat), jnp.float32),
            pltpu.SemaphoreType.DMA,
            pltpu.SemaphoreType.DMA,
            pltpu.SemaphoreType.DMA,
            pltpu.SemaphoreType.DMA,
            pltpu.SemaphoreType.DMA,
            pltpu.SemaphoreType.DMA,
        ],
    )
    def scatter(x_hbm, row_hbm, col_hbm, out_hbm,
                acc, ridx, cidx, gbuf, gbuf1,
                sem, gsem1, ssem0, ssem1, isem0, isem1):
        c = lax.axis_index("c")
        s = lax.axis_index("s")
        wid = c * _NS + s

        zero = jnp.zeros((16,), jnp.float32)

        def zrow(i, _):
            def zcol(j, _):
                gbuf[i, pl.ds(j * 16, 16)] = zero
                return 0
            return lax.fori_loop(0, n_feat // 16, zcol, 0)

        lax.fori_loop(0, _CHUNK, zrow, 0)
        row_base = s * rows_per_tile
        for k in range(rows_per_tile // _CHUNK):
            pltpu.sync_copy(gbuf, acc.at[pl.ds(row_base + k * _CHUNK, _CHUNK)])

        isems = (isem0, isem1)

        def fetch_idx(sup):
            b = sup % 2
            return (
                pltpu.async_copy(row_hbm.at[wid, pl.ds(sup * _SUP, _SUP)],
                                 ridx.at[b], isems[b]),
                pltpu.async_copy(col_hbm.at[wid, pl.ds(sup * _SUP, _SUP)],
                                 cidx.at[b], isems[b]),
            )

        pend = fetch_idx(0)
        plsc.subcore_barrier()

        for sup in range(n_sup):
            b = sup % 2
            for p in pend:
                p.wait()
            if sup + 1 < n_sup:
                pend = fetch_idx(sup + 1)

            def chunk_pair(k, _, b=b):
                i0 = 2 * k
                ga = pltpu.async_copy(x_hbm.at[cidx.at[b, i0]], gbuf, sem)
                gb = pltpu.async_copy(x_hbm.at[cidx.at[b, i0 + 1]], gbuf1,
                                      gsem1)
                ga.wait()
                sa = pltpu.async_copy(gbuf, acc.at[ridx.at[b, i0]], ssem0,
                                      add=True)
                gb.wait()
                sb = pltpu.async_copy(gbuf1, acc.at[ridx.at[b, i0 + 1]],
                                      ssem1, add=True)
                sa.wait()
                sb.wait()
                return 0

            lax.fori_loop(0, _SUP // 2, chunk_pair, 0)
        plsc.subcore_barrier()

        for k in range(rows_per_tile // _CHUNK):
            r0 = row_base + k * _CHUNK
            pltpu.sync_copy(acc.at[pl.ds(r0, _CHUNK)],
                            out_hbm.at[c, pl.ds(r0, _CHUNK)])

    return scatter


@functools.lru_cache(maxsize=None)
def _make_combine(n_nodes, n_feat, blk):
    def body(p_ref, w_ref, b_ref, o_ref):
        agg = p_ref[0] + p_ref[1]
        o_ref[...] = jnp.maximum(
            jnp.dot(agg, w_ref[...], preferred_element_type=jnp.float32)
            + b_ref[...], 0.0)

    return pl.pallas_call(
        body,
        grid=(n_nodes // blk,),
        in_specs=[
            pl.BlockSpec((2, blk, n_feat), lambda i: (0, i, 0)),
            pl.BlockSpec((n_feat, n_feat), lambda i: (0, 0)),
            pl.BlockSpec((1, n_feat), lambda i: (0, 0)),
        ],
        out_specs=pl.BlockSpec((blk, n_feat), lambda i: (i, 0)),
        out_shape=jax.ShapeDtypeStruct((n_nodes, n_feat), jnp.float32),
    )


def kernel(x, edge_index, weight, bias):
    n_nodes, in_feat = x.shape
    n_edges = edge_index.shape[1]
    ei = edge_index.astype(jnp.int32)
    rows_per_tile = -(-(n_nodes + 1) // (_NS * _CHUNK)) * _CHUNK
    n_pad = rows_per_tile * _NS
    edges_per_tile = -(-n_edges // (_NW * _SUP * _CHUNK)) * _SUP * _CHUNK
    n_edges_pad = edges_per_tile * _NW
    rows, cols = ei[0], ei[1]
    if n_edges_pad != n_edges:
        pad = n_edges_pad - n_edges
        rows = jnp.concatenate([rows, jnp.full((pad,), n_nodes, jnp.int32)])
        cols = jnp.concatenate([cols, jnp.zeros((pad,), jnp.int32)])
    n_chunks = edges_per_tile // _CHUNK
    row3 = rows.reshape(_NW, n_chunks, _CHUNK)
    col3 = cols.reshape(_NW, n_chunks, _CHUNK)
    partials = _make_scatter(n_pad, in_feat, n_edges_pad)(x, row3, col3)
    return _make_combine(n_nodes, weight.shape[1], 2000)(
        partials, weight, bias.reshape(1, -1))
